# baseline (speedup 1.0000x reference)
import jax
import jax.numpy as jnp
from jax.experimental import pallas as pl
from jax.experimental.pallas import tpu as pltpu

_TB = 4


def _fwc_body(s_ref, wt_ref, uft_ref, out_ref, uw_ref):
    e_total = wt_ref.shape[0]
    j = pl.program_id(1)

    @pl.when(j == 0)
    def _():
        for e in range(e_total):
            uw_ref[e] = jnp.dot(
                wt_ref[e], uft_ref[...], preferred_element_type=jnp.float32
            )

    base = (pl.program_id(0) * pl.num_programs(1) + j) * _TB
    for t in range(_TB):
        out_ref[t] = uw_ref[s_ref[base + t]]


def kernel(U, W, node_attributes):
    M, N1, N2, N3, K = U.shape
    E, _, C = W.shape
    B = node_attributes.shape[0]
    X = M * N1 * N2 * N3

    uft = U.reshape(X, K).T.astype(jnp.float32)
    wt = W.transpose(0, 2, 1).astype(jnp.float32)
    species = jnp.argmax(node_attributes, axis=1).astype(jnp.int32)

    nb = B // (2 * _TB)
    out = pl.pallas_call(
        _fwc_body,
        out_shape=jax.ShapeDtypeStruct((B, C, X), jnp.float32),
        grid_spec=pltpu.PrefetchScalarGridSpec(
            num_scalar_prefetch=1,
            grid=(2, nb),
            in_specs=[
                pl.BlockSpec((E, C, K), lambda i, j, s: (0, 0, 0)),
                pl.BlockSpec((K, X), lambda i, j, s: (0, 0)),
            ],
            out_specs=pl.BlockSpec((_TB, C, X), lambda i, j, s: (i * nb + j, 0, 0)),
            scratch_shapes=[pltpu.VMEM((E, C, X), jnp.float32)],
        ),
        compiler_params=pltpu.CompilerParams(
            dimension_semantics=("parallel", "arbitrary"),
            vmem_limit_bytes=48 * 1024 * 1024,
        ),
        name="fwc_gather",
    )(species, wt, uft)
    return out.reshape(B, C, M, N1, N2, N3)


# speedup vs baseline: 1.0054x; 1.0054x over previous
import jax
import jax.numpy as jnp
from jax.experimental import pallas as pl
from jax.experimental.pallas import tpu as pltpu

_TB = 2
_NBUF = 6


def _fwc_body(s_ref, wt_ref, uft_ref, out_hbm, uw_ref, stage_ref, sems):
    e_total = wt_ref.shape[0]
    c_dim = uw_ref.shape[1]
    x_dim = uw_ref.shape[2]
    nb = pl.num_programs(0)
    j = pl.program_id(0)

    @pl.when(j == 0)
    def _():
        for e in range(e_total):
            uw_ref[e] = jnp.dot(
                wt_ref[e], uft_ref[...], preferred_element_type=jnp.float32
            )

    slot = jax.lax.rem(j, _NBUF)

    @pl.when(j >= _NBUF)
    def _():
        jprev = j - _NBUF
        pltpu.make_async_copy(
            stage_ref.at[slot],
            out_hbm.at[pl.ds(jprev * _TB, _TB)],
            sems.at[slot],
        ).wait()

    half = x_dim // 2
    for t in range(_TB):
        e = s_ref[j * _TB + t]
        stage_ref[slot, t, :, :half] = uw_ref[e, :, :half]
        stage_ref[slot, t, :, half:] = uw_ref[e, :, half:]

    pltpu.make_async_copy(
        stage_ref.at[slot],
        out_hbm.at[pl.ds(j * _TB, _TB)],
        sems.at[slot],
    ).start()

    @pl.when(j == nb - 1)
    def _():
        for d in range(_NBUF):
            jd = j - d
            sd = jax.lax.rem(jd, _NBUF)

            @pl.when(jd >= 0)
            def _():
                pltpu.make_async_copy(
                    stage_ref.at[sd],
                    out_hbm.at[pl.ds(jd * _TB, _TB)],
                    sems.at[sd],
                ).wait()


def kernel(U, W, node_attributes):
    M, N1, N2, N3, K = U.shape
    E, _, C = W.shape
    B = node_attributes.shape[0]
    X = M * N1 * N2 * N3

    uft = U.reshape(X, K).T.astype(jnp.float32)
    wt = W.transpose(0, 2, 1).astype(jnp.float32)
    species = jnp.argmax(node_attributes, axis=1).astype(jnp.int32)

    nb = B // _TB
    out = pl.pallas_call(
        _fwc_body,
        out_shape=jax.ShapeDtypeStruct((B, C, X), jnp.float32),
        grid_spec=pltpu.PrefetchScalarGridSpec(
            num_scalar_prefetch=1,
            grid=(nb,),
            in_specs=[
                pl.BlockSpec((E, C, K), lambda j, s: (0, 0, 0)),
                pl.BlockSpec((K, X), lambda j, s: (0, 0)),
            ],
            out_specs=pl.BlockSpec(memory_space=pl.ANY),
            scratch_shapes=[
                pltpu.VMEM((E, C, X), jnp.float32),
                pltpu.VMEM((_NBUF, _TB, C, X), jnp.float32),
                pltpu.SemaphoreType.DMA((_NBUF,)),
            ],
        ),
        compiler_params=pltpu.CompilerParams(
            dimension_semantics=("arbitrary",),
            vmem_limit_bytes=52 * 1024 * 1024,
        ),
        name="fwc_gather_mdma",
    )(species, wt, uft)
    return out.reshape(B, C, M, N1, N2, N3)
